# baseline (speedup 1.0000x reference)
import functools

import jax
import jax.numpy as jnp
from jax.experimental import pallas as pl
from jax.experimental.pallas import tpu as pltpu

E = 16
K = 8
TB1 = 256
TB2 = 256


def _gating_kernel(x_ref, wg1_ref, bg1_ref, wg2_ref, bg2_ref,
                   w_ref, sp_ref):
    i = pl.program_id(0)
    x = x_ref[...]
    gh = jnp.maximum(
        jax.lax.dot_general(x, wg1_ref[...], (((1,), (0,)), ((), ())),
                            preferred_element_type=jnp.float32)
        + bg1_ref[...], 0.0)
    logits = jax.lax.dot_general(gh, wg2_ref[...], (((1,), (0,)), ((), ())),
                                 preferred_element_type=jnp.float32) \
        + bg2_ref[...]
    m = jnp.max(logits, axis=1, keepdims=True)
    ex = jnp.exp(logits - m)
    w = ex / jnp.sum(ex, axis=1, keepdims=True)
    w_ref[...] = w

    cnt = jnp.sum((w > 0.01).astype(jnp.float32))

    @pl.when(i == 0)
    def _():
        sp_ref[0, 0] = 0.0

    sp_ref[0, 0] += cnt


def _expert_kernel(xb_ref, w_ref, w1_ref, b1_ref, w2_ref, b2_ref,
                   wc1_ref, bc1_ref, wc2_ref, bc2_ref,
                   out_ref, selconf_ref, acc_ref, confs_ref,
                   *, tb, n_e, k_top):
    e = pl.program_id(0)
    t = pl.program_id(1)
    rows = pl.ds(t * tb, tb)

    x = xb_ref[...]
    h = jax.lax.dot_general(x, w1_ref[0], (((1,), (0,)), ((), ())),
                            preferred_element_type=jnp.float32)
    h = jnp.maximum(h + b1_ref[0], 0.0)
    o = jax.lax.dot_general(h.astype(jnp.bfloat16), w2_ref[0],
                            (((1,), (0,)), ((), ())),
                            preferred_element_type=jnp.float32) \
        + b2_ref[0]

    ch = jax.lax.dot_general(o.astype(jnp.bfloat16), wc1_ref[0],
                             (((1,), (0,)), ((), ())),
                             preferred_element_type=jnp.float32)
    ch = jnp.maximum(ch + bc1_ref[0], 0.0)
    pre = jnp.sum(ch * wc2_ref[0], axis=1, keepdims=True) + bc2_ref[0]
    conf = 1.0 / (1.0 + jnp.exp(-pre))

    wfull = w_ref[...]
    lane = jax.lax.broadcasted_iota(jnp.int32, (tb, n_e), 1)
    w_col = jnp.sum(jnp.where(lane == e, wfull, 0.0), axis=1, keepdims=True)
    rank_col = jnp.sum(
        ((wfull > w_col) | ((wfull == w_col) & (lane < e))).astype(jnp.int32),
        axis=1, keepdims=True)
    wsel = jnp.where(rank_col < k_top, w_col, 0.0)

    cw = wsel * conf
    contrib = cw * o

    first = e == 0
    prev_acc = acc_ref[rows, :]
    acc_ref[rows, :] = jnp.where(first, contrib, prev_acc + contrib)
    prev_cf = confs_ref[rows, :]
    confs_ref[rows, :] = jnp.where(
        lane == e, jnp.broadcast_to(conf, (tb, n_e)),
        jnp.where(first, 0.0, prev_cf))

    @pl.when(e == n_e - 1)
    def _():
        confs = confs_ref[rows, :]
        rank = jnp.zeros((tb, n_e), jnp.int32)
        for ep in range(n_e):
            c = wfull[:, ep:ep + 1]
            rank += ((wfull < c) | ((wfull == c) & (lane > ep))
                     ).astype(jnp.int32)
        mask = rank < k_top
        den = jnp.sum(jnp.where(mask, wfull, 0.0) * confs,
                      axis=1, keepdims=True) + 1e-6
        out_ref[...] = acc_ref[rows, :] / den
        cols = [jnp.sum(jnp.where(rank == kk, confs, 0.0),
                        axis=1, keepdims=True) for kk in range(k_top)]
        selconf_ref[...] = jnp.concatenate(cols, axis=1)


def kernel(x, W1, b1, W2, b2, Wg1, bg1, Wg2, bg2, Wc1, bc1, Wc2, bc2):
    T, D = x.shape
    n_e, _, H = W1.shape
    CH = Wc1.shape[2]

    x16 = x.astype(jnp.bfloat16)
    W1b = W1.astype(jnp.bfloat16)
    W2b = W2.astype(jnp.bfloat16)
    Wc1b = Wc1.astype(jnp.bfloat16)
    bg1r = bg1.reshape(1, H)
    bg2r = bg2.reshape(1, n_e)
    b1r = b1.reshape(n_e, 1, H)
    b2r = b2.reshape(n_e, 1, D)
    bc1r = bc1.reshape(n_e, 1, CH)
    Wc2r = Wc2.reshape(n_e, 1, CH)
    bc2r = bc2.reshape(n_e, 1, 1)

    weights, sp = pl.pallas_call(
        _gating_kernel,
        grid=(T // TB1,),
        in_specs=[
            pl.BlockSpec((TB1, D), lambda i: (i, 0)),
            pl.BlockSpec((D, H), lambda i: (0, 0)),
            pl.BlockSpec((1, H), lambda i: (0, 0)),
            pl.BlockSpec((H, n_e), lambda i: (0, 0)),
            pl.BlockSpec((1, n_e), lambda i: (0, 0)),
        ],
        out_specs=[
            pl.BlockSpec((TB1, n_e), lambda i: (i, 0)),
            pl.BlockSpec(memory_space=pltpu.SMEM),
        ],
        out_shape=[
            jax.ShapeDtypeStruct((T, n_e), jnp.float32),
            jax.ShapeDtypeStruct((1, 1), jnp.float32),
        ],
        compiler_params=pltpu.CompilerParams(
            dimension_semantics=("arbitrary",)),
    )(x, Wg1, bg1r, Wg2, bg2r)

    nt = T // TB2
    body = functools.partial(_expert_kernel, tb=TB2, n_e=n_e, k_top=K)
    out, sel_conf = pl.pallas_call(
        body,
        grid=(n_e, nt),
        in_specs=[
            pl.BlockSpec((TB2, D), lambda e, t: (t, 0)),
            pl.BlockSpec((TB2, n_e), lambda e, t: (t, 0)),
            pl.BlockSpec((1, D, H), lambda e, t: (e, 0, 0)),
            pl.BlockSpec((1, 1, H), lambda e, t: (e, 0, 0)),
            pl.BlockSpec((1, H, D), lambda e, t: (e, 0, 0)),
            pl.BlockSpec((1, 1, D), lambda e, t: (e, 0, 0)),
            pl.BlockSpec((1, D, CH), lambda e, t: (e, 0, 0)),
            pl.BlockSpec((1, 1, CH), lambda e, t: (e, 0, 0)),
            pl.BlockSpec((1, 1, CH), lambda e, t: (e, 0, 0)),
            pl.BlockSpec((1, 1, 1), lambda e, t: (e, 0, 0)),
        ],
        out_specs=[
            pl.BlockSpec((TB2, D), lambda e, t: (t, 0)),
            pl.BlockSpec((TB2, K), lambda e, t: (t, 0)),
        ],
        out_shape=[
            jax.ShapeDtypeStruct((T, D), jnp.float32),
            jax.ShapeDtypeStruct((T, K), jnp.float32),
        ],
        scratch_shapes=[
            pltpu.VMEM((T, D), jnp.float32),
            pltpu.VMEM((T, n_e), jnp.float32),
        ],
        compiler_params=pltpu.CompilerParams(
            dimension_semantics=("arbitrary", "arbitrary")),
    )(x16, weights, W1b, b1r, W2b, b2r, Wc1b, bc1r, Wc2r, bc2r)

    sparsity = jnp.reshape(sp, ()) / (T * n_e)
    return (out, weights, sel_conf, sparsity)


# speedup vs baseline: 1.0685x; 1.0685x over previous
import functools

import jax
import jax.numpy as jnp
from jax.experimental import pallas as pl
from jax.experimental.pallas import tpu as pltpu

E = 16
K = 8
TB1 = 512
TB2 = 512


def _gating_kernel(x_ref, wg1_ref, bg1_ref, wg2_ref, bg2_ref,
                   w_ref, sp_ref):
    i = pl.program_id(0)
    x = x_ref[...]
    gh = jnp.maximum(
        jax.lax.dot_general(x, wg1_ref[...], (((1,), (0,)), ((), ())),
                            preferred_element_type=jnp.float32)
        + bg1_ref[...], 0.0)
    logits = jax.lax.dot_general(gh, wg2_ref[...], (((1,), (0,)), ((), ())),
                                 preferred_element_type=jnp.float32) \
        + bg2_ref[...]
    m = jnp.max(logits, axis=1, keepdims=True)
    ex = jnp.exp(logits - m)
    w = ex / jnp.sum(ex, axis=1, keepdims=True)
    w_ref[...] = w

    cnt = jnp.sum((w > 0.01).astype(jnp.float32))

    @pl.when(i == 0)
    def _():
        sp_ref[0, 0] = 0.0

    sp_ref[0, 0] += cnt


def _expert_kernel(xb_ref, w_ref, w1_ref, b1_ref, w2_ref, b2_ref,
                   wc1_ref, bc1_ref, wc2_ref, bc2_ref,
                   out_ref, selconf_ref, acc_ref, confs_ref,
                   *, tb, n_e, k_top):
    e = pl.program_id(0)
    t = pl.program_id(1)
    rows = pl.ds(t * tb, tb)

    x = xb_ref[...]
    h = jax.lax.dot_general(x, w1_ref[0], (((1,), (0,)), ((), ())),
                            preferred_element_type=jnp.float32)
    h = jnp.maximum(h + b1_ref[0], 0.0)
    o = jax.lax.dot_general(h.astype(jnp.bfloat16), w2_ref[0],
                            (((1,), (0,)), ((), ())),
                            preferred_element_type=jnp.float32) \
        + b2_ref[0]

    ch = jax.lax.dot_general(o.astype(jnp.bfloat16), wc1_ref[0],
                             (((1,), (0,)), ((), ())),
                             preferred_element_type=jnp.float32)
    ch = jnp.maximum(ch + bc1_ref[0], 0.0)
    pre = jnp.sum(ch * wc2_ref[0], axis=1, keepdims=True) + bc2_ref[0]
    conf = 1.0 / (1.0 + jnp.exp(-pre))

    wfull = w_ref[...]
    lane = jax.lax.broadcasted_iota(jnp.int32, (tb, n_e), 1)
    w_col = jnp.sum(jnp.where(lane == e, wfull, 0.0), axis=1, keepdims=True)
    rank_col = jnp.sum(
        ((wfull > w_col) | ((wfull == w_col) & (lane < e))).astype(jnp.int32),
        axis=1, keepdims=True)
    wsel = jnp.where(rank_col < k_top, w_col, 0.0)

    cw = wsel * conf
    contrib = cw * o

    first = e == 0
    prev_acc = acc_ref[rows, :]
    acc_ref[rows, :] = jnp.where(first, contrib, prev_acc + contrib)
    prev_cf = confs_ref[rows, :]
    confs_ref[rows, :] = jnp.where(
        lane == e, jnp.broadcast_to(conf, (tb, n_e)),
        jnp.where(first, 0.0, prev_cf))

    @pl.when(e == n_e - 1)
    def _():
        confs = confs_ref[rows, :]
        rank = jnp.zeros((tb, n_e), jnp.int32)
        for ep in range(n_e):
            c = wfull[:, ep:ep + 1]
            rank += ((wfull < c) | ((wfull == c) & (lane > ep))
                     ).astype(jnp.int32)
        mask = rank < k_top
        den = jnp.sum(jnp.where(mask, wfull, 0.0) * confs,
                      axis=1, keepdims=True) + 1e-6
        out_ref[...] = acc_ref[rows, :] / den
        cols = [jnp.sum(jnp.where(rank == kk, confs, 0.0),
                        axis=1, keepdims=True) for kk in range(k_top)]
        selconf_ref[...] = jnp.concatenate(cols, axis=1)


def kernel(x, W1, b1, W2, b2, Wg1, bg1, Wg2, bg2, Wc1, bc1, Wc2, bc2):
    T, D = x.shape
    n_e, _, H = W1.shape
    CH = Wc1.shape[2]

    x16 = x.astype(jnp.bfloat16)
    W1b = W1.astype(jnp.bfloat16)
    W2b = W2.astype(jnp.bfloat16)
    Wc1b = Wc1.astype(jnp.bfloat16)
    bg1r = bg1.reshape(1, H)
    bg2r = bg2.reshape(1, n_e)
    b1r = b1.reshape(n_e, 1, H)
    b2r = b2.reshape(n_e, 1, D)
    bc1r = bc1.reshape(n_e, 1, CH)
    Wc2r = Wc2.reshape(n_e, 1, CH)
    bc2r = bc2.reshape(n_e, 1, 1)

    weights, sp = pl.pallas_call(
        _gating_kernel,
        grid=(T // TB1,),
        in_specs=[
            pl.BlockSpec((TB1, D), lambda i: (i, 0)),
            pl.BlockSpec((D, H), lambda i: (0, 0)),
            pl.BlockSpec((1, H), lambda i: (0, 0)),
            pl.BlockSpec((H, n_e), lambda i: (0, 0)),
            pl.BlockSpec((1, n_e), lambda i: (0, 0)),
        ],
        out_specs=[
            pl.BlockSpec((TB1, n_e), lambda i: (i, 0)),
            pl.BlockSpec(memory_space=pltpu.SMEM),
        ],
        out_shape=[
            jax.ShapeDtypeStruct((T, n_e), jnp.float32),
            jax.ShapeDtypeStruct((1, 1), jnp.float32),
        ],
        compiler_params=pltpu.CompilerParams(
            dimension_semantics=("arbitrary",)),
    )(x, Wg1, bg1r, Wg2, bg2r)

    nt = T // TB2
    body = functools.partial(_expert_kernel, tb=TB2, n_e=n_e, k_top=K)
    out, sel_conf = pl.pallas_call(
        body,
        grid=(n_e, nt),
        in_specs=[
            pl.BlockSpec((TB2, D), lambda e, t: (t, 0)),
            pl.BlockSpec((TB2, n_e), lambda e, t: (t, 0)),
            pl.BlockSpec((1, D, H), lambda e, t: (e, 0, 0)),
            pl.BlockSpec((1, 1, H), lambda e, t: (e, 0, 0)),
            pl.BlockSpec((1, H, D), lambda e, t: (e, 0, 0)),
            pl.BlockSpec((1, 1, D), lambda e, t: (e, 0, 0)),
            pl.BlockSpec((1, D, CH), lambda e, t: (e, 0, 0)),
            pl.BlockSpec((1, 1, CH), lambda e, t: (e, 0, 0)),
            pl.BlockSpec((1, 1, CH), lambda e, t: (e, 0, 0)),
            pl.BlockSpec((1, 1, 1), lambda e, t: (e, 0, 0)),
        ],
        out_specs=[
            pl.BlockSpec((TB2, D), lambda e, t: (t, 0)),
            pl.BlockSpec((TB2, K), lambda e, t: (t, 0)),
        ],
        out_shape=[
            jax.ShapeDtypeStruct((T, D), jnp.float32),
            jax.ShapeDtypeStruct((T, K), jnp.float32),
        ],
        scratch_shapes=[
            pltpu.VMEM((T, D), jnp.float32),
            pltpu.VMEM((T, n_e), jnp.float32),
        ],
        compiler_params=pltpu.CompilerParams(
            dimension_semantics=("arbitrary", "arbitrary")),
    )(x16, weights, W1b, b1r, W2b, b2r, Wc1b, bc1r, Wc2r, bc2r)

    sparsity = jnp.reshape(sp, ()) / (T * n_e)
    return (out, weights, sel_conf, sparsity)
